# Initial kernel scaffold; baseline (speedup 1.0000x reference)
import jax
import jax.numpy as jnp
from jax.experimental import pallas as pl


def kernel(pos, timesteps, params):
    raise NotImplementedError("write your pallas kernel here")


# speedup vs baseline: 24.4761x; 24.4761x over previous
import numpy as np
import jax
import jax.numpy as jnp
from jax.experimental import pallas as pl

_B = 8
_NG = 8
_NN = 256
_SIZE = 32
_D = 32
_NL = 5
_TI = 64


def _goal_pos():
    gi = np.linspace(0, _SIZE * _SIZE - 1, _NG).astype(np.int64)
    gx = (gi % _SIZE).astype(np.float32)
    gy = (gi // _SIZE).astype(np.float32)
    return np.stack([gx / _SIZE * 2 - 1, gy / _SIZE * 2 - 1], -1).astype(np.float32)


_GOAL_POS = _goal_pos()


def _sigmoid(x):
    return 1.0 / (1.0 + jnp.exp(-x))


def _silu(x):
    return x * _sigmoid(x)


def _fwd_kernel(p0_ref, te_ref, h0_ref, wxi_ref, wxj_ref, wt_ref, wrad_ref,
                m1b_ref, m2_ref, m2b_ref, aw_ref, ab_ref, p1_ref, p1b_ref,
                p2w_ref, p2b_ref, n1h_ref, n1a_ref, n1b_ref, n2_ref, n2b_ref,
                tw_ref, tb_ref, out_ref):
    h = h0_ref[...]
    p = p0_ref[0]
    te = te_ref[0]
    px = p[:, 0:1]
    py = p[:, 1:2]
    pxr = px.reshape(1, _NN)
    pyr = py.reshape(1, _NN)

    for l in range(_NL):
        ai_all = h @ wxi_ref[l] + (te @ wt_ref[l]) + m1b_ref[l]
        aj_all = h @ wxj_ref[l]
        wrad3 = wrad_ref[l].reshape(1, 1, _D)
        m2w = m2_ref[l]
        m2b = m2b_ref[l]
        aw = aw_ref[l]
        ab = ab_ref[l][0]
        p1w = p1_ref[l]
        p1b = p1b_ref[l]
        p2w = p2w_ref[l]
        p2b = p2b_ref[l][0]

        aggr_h_parts = []
        apx_parts = []
        apy_parts = []
        for c in range(_NN // _TI):
            i0 = c * _TI
            ai_c = ai_all[i0:i0 + _TI]
            px_c = px[i0:i0 + _TI]
            py_c = py[i0:i0 + _TI]
            dx = px_c - pxr
            dy = py_c - pyr
            radial = dx * dx + dy * dy
            norm = jnp.sqrt(radial) + 1e-6
            dxn = dx / norm
            dyn = dy / norm
            pre = (ai_c[:, None, :] + aj_all[None, :, :]
                   + radial[:, :, None] * wrad3)
            m = _silu(pre).reshape(_TI * _NN, _D)
            m = _silu(m @ m2w + m2b)
            att = _sigmoid(m @ aw + ab)
            m = m * att
            t1 = _silu(m @ p1w + p1b)
            pv = t1 @ p2w + p2b
            pv2 = pv.reshape(_TI, _NN)
            aggr_h_parts.append(m.reshape(_TI, _NN, _D).sum(axis=1))
            apx_parts.append(jnp.sum(dxn * pv2, axis=1, keepdims=True))
            apy_parts.append(jnp.sum(dyn * pv2, axis=1, keepdims=True))
        aggr_h = jnp.concatenate(aggr_h_parts, axis=0)
        apx = jnp.concatenate(apx_parts, axis=0)
        apy = jnp.concatenate(apy_parts, axis=0)

        if l < _NL - 1:
            u = h @ n1h_ref[l] + aggr_h @ n1a_ref[l] + n1b_ref[l]
            u = _silu(u)
            u = u @ n2_ref[l] + n2b_ref[l]
            u = jnp.where(u >= 0, u, 0.01 * u)
            h = h + u
            te = _silu(te @ tw_ref[l] + tb_ref[l])

        px = px + apx * (1.0 / _NN)
        py = py + apy * (1.0 / _NN)
        pxr = px.reshape(1, _NN)
        pyr = py.reshape(1, _NN)

    out_ref[0] = jnp.concatenate([px[_NG:], py[_NG:]], axis=1)


def kernel(pos, timesteps, params):
    goal = jnp.asarray(_GOAL_POS)
    p0 = jnp.concatenate(
        [jnp.broadcast_to(goal[None], (_B, _NG, 2)), pos], axis=1)

    t = timesteps.astype(jnp.float32)
    half = _D // 2
    freqs = jnp.exp(-np.log(10000.0) * jnp.arange(half, dtype=jnp.float32) / half)
    args = t[:, None] * freqs[None, :]
    temb = jnp.concatenate([jnp.cos(args), jnp.sin(args)], axis=-1)
    temb = temb.reshape(_B, 1, _D)

    hw = params["h_in"]["w"]
    hb = params["h_in"]["b"]
    hg = hw[:, 0] + hb
    hs = hw[:, 1] + hb
    h0 = jnp.where(jnp.arange(_NN)[:, None] < _NG, hg[None, :], hs[None, :])

    lyr = params["layers"]

    def stk(f):
        return jnp.stack([f(lp) for lp in lyr])

    def stk4(f):
        return jnp.stack([f(lp) for lp in lyr[:_NL - 1]])

    wxiT = stk(lambda lp: lp["m1"]["w"][:, 0:_D].T)
    wxjT = stk(lambda lp: lp["m1"]["w"][:, _D:2 * _D].T)
    wrad = stk(lambda lp: lp["m1"]["w"][:, 2 * _D:2 * _D + 1].T)
    wtT = stk(lambda lp: lp["m1"]["w"][:, 2 * _D + 1:].T)
    m1b = stk(lambda lp: lp["m1"]["b"][None, :])
    m2T = stk(lambda lp: lp["m2"]["w"].T)
    m2b = stk(lambda lp: lp["m2"]["b"][None, :])
    awT = stk(lambda lp: lp["a"]["w"].T)
    ab = stk(lambda lp: lp["a"]["b"][None, :])
    p1T = stk(lambda lp: lp["p1"]["w"].T)
    p1b = stk(lambda lp: lp["p1"]["b"][None, :])
    p2w = stk(lambda lp: lp["p2"]["w"].T)
    p2b = stk(lambda lp: lp["p2"]["b"][None, :])
    n1hT = stk4(lambda lp: lp["n1"]["w"][:, 0:_D].T)
    n1aT = stk4(lambda lp: lp["n1"]["w"][:, _D:].T)
    n1b = stk4(lambda lp: lp["n1"]["b"][None, :])
    n2T = stk4(lambda lp: lp["n2"]["w"].T)
    n2b = stk4(lambda lp: lp["n2"]["b"][None, :])
    tT = jnp.stack([tl["w"].T for tl in params["t_layers"]])
    tb = jnp.stack([tl["b"][None, :] for tl in params["t_layers"]])

    def full(shape):
        return pl.BlockSpec(shape, lambda b: (0,) * len(shape))

    in_specs = [
        pl.BlockSpec((1, _NN, 2), lambda b: (b, 0, 0)),
        pl.BlockSpec((1, 1, _D), lambda b: (b, 0, 0)),
        full((_NN, _D)),
        full((_NL, _D, _D)), full((_NL, _D, _D)), full((_NL, _D, _D)),
        full((_NL, 1, _D)), full((_NL, 1, _D)),
        full((_NL, _D, _D)), full((_NL, 1, _D)),
        full((_NL, _D, 1)), full((_NL, 1, 1)),
        full((_NL, _D, _D)), full((_NL, 1, _D)),
        full((_NL, _D, 1)), full((_NL, 1, 1)),
        full((_NL - 1, _D, _D)), full((_NL - 1, _D, _D)), full((_NL - 1, 1, _D)),
        full((_NL - 1, _D, _D)), full((_NL - 1, 1, _D)),
        full((_NL - 1, _D, _D)), full((_NL - 1, 1, _D)),
    ]

    out = pl.pallas_call(
        _fwd_kernel,
        grid=(_B,),
        in_specs=in_specs,
        out_specs=pl.BlockSpec((1, _NN - _NG, 2), lambda b: (b, 0, 0)),
        out_shape=jax.ShapeDtypeStruct((_B, _NN - _NG, 2), jnp.float32),
    )(p0, temb, h0, wxiT, wxjT, wtT, wrad, m1b, m2T, m2b, awT, ab,
      p1T, p1b, p2w, p2b, n1hT, n1aT, n1b, n2T, n2b, tT, tb)
    return out


# speedup vs baseline: 27.9035x; 1.1400x over previous
import numpy as np
import jax
import jax.numpy as jnp
from jax.experimental import pallas as pl

_B = 8
_NG = 8
_NN = 256
_SIZE = 32
_D = 32
_NL = 5
_TI = 64


def _goal_pos():
    gi = np.linspace(0, _SIZE * _SIZE - 1, _NG).astype(np.int64)
    gx = (gi % _SIZE).astype(np.float32)
    gy = (gi // _SIZE).astype(np.float32)
    return np.stack([gx / _SIZE * 2 - 1, gy / _SIZE * 2 - 1], -1).astype(np.float32)


_GOAL_POS = _goal_pos()


def _sigmoid(x):
    return 0.5 * jnp.tanh(0.5 * x) + 0.5


def _silu(x):
    return x * _sigmoid(x)


def _fwd_kernel(p0_ref, te_ref, h0_ref, wxi_ref, wxj_ref, wt_ref, wrad_ref,
                m1b_ref, m2_ref, m2b_ref, aw_ref, ab_ref, p1_ref, p1b_ref,
                p2w_ref, p2b_ref, n1h_ref, n1a_ref, n1b_ref, n2_ref, n2b_ref,
                tw_ref, tb_ref, out_ref):
    h = h0_ref[...]
    p = p0_ref[0]
    te = te_ref[0]
    px = p[:, 0:1]
    py = p[:, 1:2]
    pxr = px.reshape(1, _NN)
    pyr = py.reshape(1, _NN)

    for l in range(_NL):
        ai_all = h @ wxi_ref[l] + (te @ wt_ref[l]) + m1b_ref[l]
        aj_all = h @ wxj_ref[l]
        wrad3 = wrad_ref[l].reshape(1, 1, _D)
        m2w = m2_ref[l]
        m2b = m2b_ref[l]
        aw = aw_ref[l]
        ab = ab_ref[l][0]
        p1w = p1_ref[l]
        p1b = p1b_ref[l]
        p2w = p2w_ref[l]
        p2b = p2b_ref[l][0]

        aggr_h_parts = []
        apx_parts = []
        apy_parts = []
        for c in range(_NN // _TI):
            i0 = c * _TI
            ai_c = ai_all[i0:i0 + _TI]
            px_c = px[i0:i0 + _TI]
            py_c = py[i0:i0 + _TI]
            dx = px_c - pxr
            dy = py_c - pyr
            radial = dx * dx + dy * dy
            norm = jnp.sqrt(radial) + 1e-6
            dxn = dx / norm
            dyn = dy / norm
            pre = (ai_c[:, None, :] + aj_all[None, :, :]
                   + radial[:, :, None] * wrad3)
            m = _silu(pre).reshape(_TI * _NN, _D)
            m = _silu(m @ m2w + m2b)
            att = _sigmoid(m @ aw + ab)
            m = m * att
            t1 = _silu(m @ p1w + p1b)
            pv = t1 @ p2w + p2b
            pv2 = pv.reshape(_TI, _NN)
            aggr_h_parts.append(m.reshape(_TI, _NN, _D).sum(axis=1))
            apx_parts.append(jnp.sum(dxn * pv2, axis=1, keepdims=True))
            apy_parts.append(jnp.sum(dyn * pv2, axis=1, keepdims=True))
        aggr_h = jnp.concatenate(aggr_h_parts, axis=0)
        apx = jnp.concatenate(apx_parts, axis=0)
        apy = jnp.concatenate(apy_parts, axis=0)

        if l < _NL - 1:
            u = h @ n1h_ref[l] + aggr_h @ n1a_ref[l] + n1b_ref[l]
            u = _silu(u)
            u = u @ n2_ref[l] + n2b_ref[l]
            u = jnp.where(u >= 0, u, 0.01 * u)
            h = h + u
            te = _silu(te @ tw_ref[l] + tb_ref[l])

        px = px + apx * (1.0 / _NN)
        py = py + apy * (1.0 / _NN)
        pxr = px.reshape(1, _NN)
        pyr = py.reshape(1, _NN)

    out_ref[0] = jnp.concatenate([px[_NG:], py[_NG:]], axis=1)


def kernel(pos, timesteps, params):
    goal = jnp.asarray(_GOAL_POS)
    p0 = jnp.concatenate(
        [jnp.broadcast_to(goal[None], (_B, _NG, 2)), pos], axis=1)

    t = timesteps.astype(jnp.float32)
    half = _D // 2
    freqs = jnp.exp(-np.log(10000.0) * jnp.arange(half, dtype=jnp.float32) / half)
    args = t[:, None] * freqs[None, :]
    temb = jnp.concatenate([jnp.cos(args), jnp.sin(args)], axis=-1)
    temb = temb.reshape(_B, 1, _D)

    hw = params["h_in"]["w"]
    hb = params["h_in"]["b"]
    hg = hw[:, 0] + hb
    hs = hw[:, 1] + hb
    h0 = jnp.where(jnp.arange(_NN)[:, None] < _NG, hg[None, :], hs[None, :])

    lyr = params["layers"]

    def stk(f):
        return jnp.stack([f(lp) for lp in lyr])

    def stk4(f):
        return jnp.stack([f(lp) for lp in lyr[:_NL - 1]])

    wxiT = stk(lambda lp: lp["m1"]["w"][:, 0:_D].T)
    wxjT = stk(lambda lp: lp["m1"]["w"][:, _D:2 * _D].T)
    wrad = stk(lambda lp: lp["m1"]["w"][:, 2 * _D:2 * _D + 1].T)
    wtT = stk(lambda lp: lp["m1"]["w"][:, 2 * _D + 1:].T)
    m1b = stk(lambda lp: lp["m1"]["b"][None, :])
    m2T = stk(lambda lp: lp["m2"]["w"].T)
    m2b = stk(lambda lp: lp["m2"]["b"][None, :])
    awT = stk(lambda lp: lp["a"]["w"].T)
    ab = stk(lambda lp: lp["a"]["b"][None, :])
    p1T = stk(lambda lp: lp["p1"]["w"].T)
    p1b = stk(lambda lp: lp["p1"]["b"][None, :])
    p2w = stk(lambda lp: lp["p2"]["w"].T)
    p2b = stk(lambda lp: lp["p2"]["b"][None, :])
    n1hT = stk4(lambda lp: lp["n1"]["w"][:, 0:_D].T)
    n1aT = stk4(lambda lp: lp["n1"]["w"][:, _D:].T)
    n1b = stk4(lambda lp: lp["n1"]["b"][None, :])
    n2T = stk4(lambda lp: lp["n2"]["w"].T)
    n2b = stk4(lambda lp: lp["n2"]["b"][None, :])
    tT = jnp.stack([tl["w"].T for tl in params["t_layers"]])
    tb = jnp.stack([tl["b"][None, :] for tl in params["t_layers"]])

    def full(shape):
        return pl.BlockSpec(shape, lambda b: (0,) * len(shape))

    in_specs = [
        pl.BlockSpec((1, _NN, 2), lambda b: (b, 0, 0)),
        pl.BlockSpec((1, 1, _D), lambda b: (b, 0, 0)),
        full((_NN, _D)),
        full((_NL, _D, _D)), full((_NL, _D, _D)), full((_NL, _D, _D)),
        full((_NL, 1, _D)), full((_NL, 1, _D)),
        full((_NL, _D, _D)), full((_NL, 1, _D)),
        full((_NL, _D, 1)), full((_NL, 1, 1)),
        full((_NL, _D, _D)), full((_NL, 1, _D)),
        full((_NL, _D, 1)), full((_NL, 1, 1)),
        full((_NL - 1, _D, _D)), full((_NL - 1, _D, _D)), full((_NL - 1, 1, _D)),
        full((_NL - 1, _D, _D)), full((_NL - 1, 1, _D)),
        full((_NL - 1, _D, _D)), full((_NL - 1, 1, _D)),
    ]

    out = pl.pallas_call(
        _fwd_kernel,
        grid=(_B,),
        in_specs=in_specs,
        out_specs=pl.BlockSpec((1, _NN - _NG, 2), lambda b: (b, 0, 0)),
        out_shape=jax.ShapeDtypeStruct((_B, _NN - _NG, 2), jnp.float32),
    )(p0, temb, h0, wxiT, wxjT, wtT, wrad, m1b, m2T, m2b, awT, ab,
      p1T, p1b, p2w, p2b, n1hT, n1aT, n1b, n2T, n2b, tT, tb)
    return out
